# baseline (speedup 1.0000x reference)
import jax
import jax.numpy as jnp
from jax import lax
from jax.experimental import pallas as pl
from jax.experimental.pallas import tpu as pltpu
from jax.experimental.pallas import tpu_sc as plsc

N = 10000
E = 320000
NC = 2
NS = 16
NW = NC * NS
IB = 125
EXP = 640

F32 = jnp.float32
I32 = jnp.int32


def _m8(v):
    return pl.multiple_of(v, 8)


def _export_base(sid):
    return _m8(jnp.minimum(sid * EXP, N - EXP))


def _node_proj_body(x_ref, batch_ref, wrc_ref, w2x_ref, u_ref, w2u_ref,
                    b2a_ref, xr_ref, xc_ref, base2_ref):
    xb = x_ref[...]
    xw = jnp.dot(xb, wrc_ref[...], preferred_element_type=F32)
    xr_ref[...] = xw[:, :32]
    xc_ref[...] = xw[:, 32:]
    oh = (batch_ref[...] == lax.broadcasted_iota(I32, (1, 16), 1)).astype(F32)
    uw = jnp.dot(u_ref[...], w2u_ref[...], preferred_element_type=F32)
    base2_ref[...] = (jnp.dot(xb, w2x_ref[...], preferred_element_type=F32)
                      + jnp.dot(oh, uw, preferred_element_type=F32)
                      + b2a_ref[...])


def _gather_body(xr_hbm, xc_hbm, row2d_hbm, col2d_hbm, tr_hbm, tc_hbm,
                 idxr_v, idxc_v, bufr_v, bufc_v, sem1, sem2):
    wid = lax.axis_index("s") * NC + lax.axis_index("c")
    rows_per_tile = (E // NW) // IB
    k = 8
    for it in range(rows_per_tile // k):
        roff = _m8(wid * rows_per_tile + it * k)
        pltpu.sync_copy(row2d_hbm.at[pl.ds(roff, k)], idxr_v)
        pltpu.sync_copy(col2d_hbm.at[pl.ds(roff, k)], idxc_v)
        cps = []
        for b in range(k):
            cps.append(pltpu.async_copy(
                xr_hbm.at[idxr_v.at[b]], bufr_v.at[pl.ds(b * IB, IB)], sem1))
            cps.append(pltpu.async_copy(
                xc_hbm.at[idxc_v.at[b]], bufc_v.at[pl.ds(b * IB, IB)], sem2))
        for cp in cps:
            cp.wait()
        eoff = _m8(roff * IB)
        pltpu.sync_copy(bufr_v, tr_hbm.at[pl.ds(eoff, k * IB)])
        pltpu.sync_copy(bufc_v, tc_hbm.at[pl.ds(eoff, k * IB)])


def _edge_mlp_body(tr_ref, tc_ref, ea8_ref, w1ebd_ref, b1a_ref, w1b_ref,
                   b1b_ref, h_ref):
    ep = jnp.dot(ea8_ref[...], w1ebd_ref[...], preferred_element_type=F32)
    nsub = ep.shape[0]
    for i in range(8):
        sl = pl.ds(i * nsub, nsub)
        pre = (tr_ref[sl, :] + tc_ref[sl, :]
               + ep[:, i * 32:(i + 1) * 32] + b1a_ref[...])
        h1 = jnp.maximum(pre, 0.0)
        h_ref[sl, :] = jnp.maximum(
            jnp.dot(h1, w1b_ref[...], preferred_element_type=F32)
            + b1b_ref[...], 0.0)


def _scatter_sum_body(h_hbm, col2d_hbm, ones_hbm, z32_hbm, z8_hbm,
                      sum_hbm, cnt_hbm,
                      idx_v, h_v, ones_v, c_stage, ex_stage, acc_sh, cnt_sh):
    sc = lax.axis_index("c")
    sid = lax.axis_index("s")
    base = _export_base(sid)
    pltpu.sync_copy(z32_hbm, ex_stage)
    pltpu.sync_copy(ex_stage, acc_sh.at[pl.ds(base, EXP)])
    pltpu.sync_copy(z8_hbm, c_stage)
    pltpu.sync_copy(c_stage, cnt_sh.at[pl.ds(base, EXP)])
    pltpu.sync_copy(ones_hbm, ones_v)
    plsc.subcore_barrier()
    rows_per_tile = (E // NW) // IB
    k = 8
    for it in range(rows_per_tile // k):
        roff = _m8((sc * NS + sid) * rows_per_tile + it * k)
        pltpu.sync_copy(col2d_hbm.at[pl.ds(roff, k)], idx_v)
        pltpu.sync_copy(h_hbm.at[pl.ds(_m8(roff * IB), k * IB)], h_v)
        for b in range(k):
            pltpu.sync_copy(h_v.at[pl.ds(b * IB, IB)],
                            acc_sh.at[idx_v.at[b]], add=True)
            pltpu.sync_copy(ones_v, cnt_sh.at[idx_v.at[b]], add=True)
    plsc.subcore_barrier()
    sl = pl.ds(base, EXP)
    pltpu.sync_copy(acc_sh.at[sl], ex_stage)
    pltpu.sync_copy(ex_stage, sum_hbm.at[sc, sl])
    pltpu.sync_copy(cnt_sh.at[sl], c_stage)
    pltpu.sync_copy(c_stage, cnt_hbm.at[sc, sl])


def _scatter_max_body(h_hbm, col_hbm, z128_hbm, maxp_hbm,
                      acc_v, idx0_v, idx1_v, h0_v, h1_v, dup_v, sem0, sem1):
    sc = lax.axis_index("c")
    sid = lax.axis_index("s")
    wid = sid * NC + sc
    g = wid % 4
    chk = wid // 4
    pltpu.sync_copy(z128_hbm, acc_v)
    iota16 = lax.broadcasted_iota(I32, (16,), 0)
    CH = 400
    per_chunk = E // 8
    ebase = chk * per_chunk
    nch = per_chunk // CH

    def fetch(it, idxr, hr, sem):
        off = _m8(ebase + it * CH)
        pltpu.async_copy(col_hbm.at[pl.ds(off, CH)], idxr, sem)
        pltpu.async_copy(h_hbm.at[pl.ds(off, CH), pl.ds(g * 8, 8)], hr, sem)

    def waitpair(it, idxr, hr, sem):
        off = _m8(ebase + it * CH)
        pltpu.make_async_copy(col_hbm.at[pl.ds(off, CH)], idxr, sem).wait()
        pltpu.make_async_copy(
            h_hbm.at[pl.ds(off, CH), pl.ds(g * 8, 8)], hr, sem).wait()

    def process(idxr, hr):
        def group_step(gi, carry2):
            e0 = gi * 16
            cols = idxr[pl.ds(e0, 16)]
            evec = e0 + iota16
            plsc.store_scatter(dup_v, [cols], iota16)
            rb = plsc.load_gather(dup_v, [cols])
            bv = jnp.right_shift(cols * 53688, 25)
            rowv = cols - bv * 625
            colb = jnp.left_shift(bv, 3)
            jvs = [jnp.full((16,), j, I32) for j in range(8)]
            cvs = [colb + j for j in range(8)]
            hvs = [plsc.load_gather(hr, [evec, jv]) for jv in jvs]
            curs = [plsc.load_gather(acc_v, [rowv, cv]) for cv in cvs]
            news = [jnp.maximum(c, v) for c, v in zip(curs, hvs)]
            for cv, nv in zip(cvs, news):
                plsc.store_scatter(acc_v, [rowv, cv], nv)
            ndup = plsc.all_reduce_population_count(rb != iota16)
            hasdup = jnp.squeeze(lax.slice(ndup, [0], [1]))

            @pl.when(hasdup > 0)
            def _fixup():
                for j in range(8):
                    cv = cvs[j]
                    hv = hvs[j]

                    def retry_cond(nbad):
                        return nbad > 0

                    def retry_body(nbad):
                        cur = plsc.load_gather(acc_v, [rowv, cv])
                        m = cur < hv
                        plsc.store_scatter(acc_v, [rowv, cv],
                                           jnp.maximum(cur, hv), mask=m)
                        cur2 = plsc.load_gather(acc_v, [rowv, cv])
                        return jnp.sum((cur2 < hv).astype(I32))

                    lax.while_loop(retry_cond, retry_body, jnp.int32(1))

            return carry2

        lax.fori_loop(0, CH // 16, group_step, 0)

    fetch(0, idx0_v, h0_v, sem0)

    def pair_step(itp, carry):
        it0 = 2 * itp
        it1 = it0 + 1
        waitpair(it0, idx0_v, h0_v, sem0)
        fetch(it1, idx1_v, h1_v, sem1)
        process(idx0_v, h0_v)
        waitpair(it1, idx1_v, h1_v, sem1)

        @pl.when(itp + 1 < nch // 2)
        def _pf():
            fetch(it0 + 2, idx0_v, h0_v, sem0)

        process(idx1_v, h1_v)
        return carry

    lax.fori_loop(0, nch // 2, pair_step, 0)
    pltpu.sync_copy(acc_v, maxp_hbm.at[chk, g])


def _final_body(sum_ref, cnt_ref, maxp_ref, base2_ref, w2m_ref, wsel_ref,
                w2b_ref, b2b_ref, out_ref):
    i = pl.program_id(0)
    s = sum_ref[0] + sum_ref[1]
    c8 = cnt_ref[0] + cnt_ref[1]
    cnt = jnp.maximum(c8[:, 0:1], 1.0)
    mean = s / cnt
    meanw = jnp.dot(mean, w2m_ref[...], preferred_element_type=F32)
    ms = []
    for g in range(4):
        m = maxp_ref[0, g]
        for c in range(1, 8):
            m = jnp.maximum(m, maxp_ref[c, g])
        ms.append(m)
    for k in range(8):
        kk = 8 * i + k
        mxw = jnp.dot(ms[0], wsel_ref[0, kk], preferred_element_type=F32)
        for g in range(1, 4):
            mxw = mxw + jnp.dot(ms[g], wsel_ref[g, kk],
                                preferred_element_type=F32)
        sl = pl.ds(k * 625, 625)
        h2 = jnp.maximum(
            base2_ref[sl, :] + meanw[k * 625:(k + 1) * 625, :] + mxw, 0.0)
        out_ref[sl, :] = jnp.maximum(
            jnp.dot(h2, w2b_ref[...], preferred_element_type=F32)
            + b2b_ref[...], 0.0)


def kernel(x, edge_index, edge_attr, u, batch,
           W1a, b1a, W1b, b1b, W2a, b2a, W2b, b2b):
    row = edge_index[0].astype(I32).reshape(50, 800, 8)
    col = edge_index[1].astype(I32).reshape(50, 800, 8)
    row = row.transpose(0, 2, 1).reshape(E)
    col = col.transpose(0, 2, 1).reshape(E)
    row2d = row.reshape(E // IB, IB)
    col2d = col.reshape(E // IB, IB)
    batch2d = batch.astype(I32).reshape(N, 1)
    W_rc = jnp.concatenate([W1a[:128], W1a[128:256]], axis=1)
    W1e = W1a[256:]
    W2x = W2a[:128]
    W2mean = W2a[128:160]
    W2max = W2a[160:192]
    W2u = W2a[192:224]
    b1a2 = b1a.reshape(1, 32)
    b1b2 = b1b.reshape(1, 32)
    b2a2 = b2a.reshape(1, 32)
    b2b2 = b2b.reshape(1, 128)
    ones8 = jnp.ones((IB, 8), F32)
    z32 = jnp.zeros((EXP, 32), F32)
    z8 = jnp.zeros((EXP, 8), F32)

    BN = 2000
    full = lambda shape: pl.BlockSpec(shape, lambda i, _s=shape: tuple(0 for _ in _s))
    xr, xc, base2 = pl.pallas_call(
        _node_proj_body,
        grid=(N // BN,),
        in_specs=[
            pl.BlockSpec((BN, 128), lambda i: (i, 0)),
            pl.BlockSpec((BN, 1), lambda i: (i, 0)),
            full((128, 64)), full((128, 32)), full((16, 32)),
            full((32, 32)), full((1, 32)),
        ],
        out_specs=[pl.BlockSpec((BN, 32), lambda i: (i, 0))] * 3,
        out_shape=[jax.ShapeDtypeStruct((N, 32), F32)] * 3,
    )(x, batch2d, W_rc, W2x, u, W2u, b2a2)

    mesh = plsc.VectorSubcoreMesh(core_axis_name="c", subcore_axis_name="s",
                                  num_cores=NC, num_subcores=NS)
    sc_params = pltpu.CompilerParams(use_tc_tiling_on_sc=False,
                                     needs_layout_passes=False)
    t_r, t_c = pl.kernel(
        _gather_body,
        out_type=[jax.ShapeDtypeStruct((E, 32), F32)] * 2,
        mesh=mesh,
        compiler_params=sc_params,
        scratch_types=[
            pltpu.VMEM((8, IB), I32), pltpu.VMEM((8, IB), I32),
            pltpu.VMEM((1000, 32), F32), pltpu.VMEM((1000, 32), F32),
            pltpu.SemaphoreType.DMA, pltpu.SemaphoreType.DMA,
        ],
    )(xr, xc, row2d, col2d)

    BE = 6400
    ea8 = edge_attr.reshape(E // 8, 128)
    W1e_bd = jnp.zeros((128, 256), F32)
    for i in range(8):
        W1e_bd = W1e_bd.at[i * 16:(i + 1) * 16, i * 32:(i + 1) * 32].set(W1e)
    h = pl.pallas_call(
        _edge_mlp_body,
        grid=(E // BE,),
        in_specs=[
            pl.BlockSpec((BE, 32), lambda i: (i, 0)),
            pl.BlockSpec((BE, 32), lambda i: (i, 0)),
            pl.BlockSpec((BE // 8, 128), lambda i: (i, 0)),
            full((128, 256)), full((1, 32)), full((32, 32)), full((1, 32)),
        ],
        out_specs=pl.BlockSpec((BE, 32), lambda i: (i, 0)),
        out_shape=jax.ShapeDtypeStruct((E, 32), F32),
    )(t_r, t_c, ea8, W1e_bd, b1a2, W1b, b1b2)

    sum_part, cnt_part = pl.kernel(
        _scatter_sum_body,
        out_type=[jax.ShapeDtypeStruct((NC, N, 32), F32),
                  jax.ShapeDtypeStruct((NC, N, 8), F32)],
        mesh=mesh,
        compiler_params=sc_params,
        scratch_types=[
            pltpu.VMEM((8, IB), I32),
            pltpu.VMEM((1000, 32), F32),
            pltpu.VMEM((IB, 8), F32),
            pltpu.VMEM((EXP, 8), F32),
            pltpu.VMEM((EXP, 32), F32),
            pltpu.VMEM_SHARED((N, 32), F32),
            pltpu.VMEM_SHARED((N, 8), F32),
        ],
    )(h, col2d, ones8, z32, z8)

    z128 = jnp.zeros((N // 16, 128), F32)
    max_part = pl.kernel(
        _scatter_max_body,
        out_type=jax.ShapeDtypeStruct((8, 4, N // 16, 128), F32),
        mesh=mesh,
        compiler_params=sc_params,
        scratch_types=[
            pltpu.VMEM((N // 16, 128), F32),
            pltpu.VMEM((400,), I32), pltpu.VMEM((400,), I32),
            pltpu.VMEM((400, 8), F32), pltpu.VMEM((400, 8), F32),
            pltpu.VMEM((N,), I32),
            pltpu.SemaphoreType.DMA, pltpu.SemaphoreType.DMA,
        ],
    )(h, col, z128)

    Wsel = jnp.einsum('kb,gjo->gkbjo', jnp.eye(16, dtype=F32),
                      W2max.reshape(4, 8, 32)).reshape(4, 16, 128, 32)
    BF = 5000
    out = pl.pallas_call(
        _final_body,
        grid=(N // BF,),
        in_specs=[
            pl.BlockSpec((NC, BF, 32), lambda i: (0, i, 0)),
            pl.BlockSpec((NC, BF, 8), lambda i: (0, i, 0)),
            pl.BlockSpec((8, 4, N // 16, 128), lambda i: (0, 0, 0, 0)),
            pl.BlockSpec((BF, 32), lambda i: (i, 0)),
            full((32, 32)), full((4, 16, 128, 32)), full((32, 128)),
            full((1, 128)),
        ],
        out_specs=pl.BlockSpec((BF, 128), lambda i: (i, 0)),
        out_shape=jax.ShapeDtypeStruct((N, 128), F32),
    )(sum_part, cnt_part, max_part, base2, W2mean, Wsel, W2b, b2b2)
    return out


# speedup vs baseline: 1.2028x; 1.2028x over previous
import jax
import jax.numpy as jnp
from jax import lax
from jax.experimental import pallas as pl
from jax.experimental.pallas import tpu as pltpu
from jax.experimental.pallas import tpu_sc as plsc

N = 10000
E = 320000
NC = 2
NS = 16
NW = NC * NS
IB = 125
EXP = 640

F32 = jnp.float32
I32 = jnp.int32


def _m8(v):
    return pl.multiple_of(v, 8)


def _export_base(sid):
    return _m8(jnp.minimum(sid * EXP, N - EXP))


def _node_proj_body(x_ref, batch_ref, wrc_ref, w2x_ref, u_ref, w2u_ref,
                    b2a_ref, xr_ref, xc_ref, base2_ref):
    xb = x_ref[...]
    xw = jnp.dot(xb, wrc_ref[...], preferred_element_type=F32)
    xr_ref[...] = xw[:, :32]
    xc_ref[...] = xw[:, 32:]
    oh = (batch_ref[...] == lax.broadcasted_iota(I32, (1, 16), 1)).astype(F32)
    uw = jnp.dot(u_ref[...], w2u_ref[...], preferred_element_type=F32)
    base2_ref[...] = (jnp.dot(xb, w2x_ref[...], preferred_element_type=F32)
                      + jnp.dot(oh, uw, preferred_element_type=F32)
                      + b2a_ref[...])


def _gather_body(xr_hbm, xc_hbm, row2d_hbm, col2d_hbm, trc_hbm,
                 idxr_v, idxc_v, bufr_v, bufc_v, sem1, sem2):
    wid = lax.axis_index("s") * NC + lax.axis_index("c")
    rows_per_tile = (E // NW) // IB
    k = 8
    for it in range(rows_per_tile // k):
        roff = _m8(wid * rows_per_tile + it * k)
        pltpu.sync_copy(row2d_hbm.at[pl.ds(roff, k)], idxr_v)
        pltpu.sync_copy(col2d_hbm.at[pl.ds(roff, k)], idxc_v)
        cps = []
        for b in range(k):
            cps.append(pltpu.async_copy(
                xr_hbm.at[idxr_v.at[b]], bufr_v.at[pl.ds(b * IB, IB)], sem1))
            cps.append(pltpu.async_copy(
                xc_hbm.at[idxc_v.at[b]], bufc_v.at[pl.ds(b * IB, IB)], sem2))
        for cp in cps:
            cp.wait()

        def add_row(j, carry):
            a0 = bufr_v[j, pl.ds(0, 16)]
            b0 = bufc_v[j, pl.ds(0, 16)]
            bufr_v[j, pl.ds(0, 16)] = a0 + b0
            a1 = bufr_v[j, pl.ds(16, 16)]
            b1 = bufc_v[j, pl.ds(16, 16)]
            bufr_v[j, pl.ds(16, 16)] = a1 + b1
            return carry

        lax.fori_loop(0, k * IB, add_row, 0)
        eoff = _m8(roff * IB)
        pltpu.sync_copy(bufr_v, trc_hbm.at[pl.ds(eoff, k * IB)])


def _edge_mlp_body(trc_ref, ea8_ref, w1ebd_ref, b1a_ref, w1b_ref,
                   b1b_ref, h_ref):
    ep = jnp.dot(ea8_ref[...], w1ebd_ref[...], preferred_element_type=F32)
    nsub = ep.shape[0]
    for i in range(8):
        sl = pl.ds(i * nsub, nsub)
        pre = (trc_ref[sl, :]
               + ep[:, i * 32:(i + 1) * 32] + b1a_ref[...])
        h1 = jnp.maximum(pre, 0.0)
        h_ref[sl, :] = jnp.maximum(
            jnp.dot(h1, w1b_ref[...], preferred_element_type=F32)
            + b1b_ref[...], 0.0)


def _scatter_sum_body(h_hbm, col2d_hbm, ones_hbm, z32_hbm, z8_hbm,
                      sum_hbm, cnt_hbm,
                      idx_v, h_v, ones_v, c_stage, ex_stage, acc_sh, cnt_sh):
    sc = lax.axis_index("c")
    sid = lax.axis_index("s")
    base = _export_base(sid)
    pltpu.sync_copy(z32_hbm, ex_stage)
    pltpu.sync_copy(ex_stage, acc_sh.at[pl.ds(base, EXP)])
    pltpu.sync_copy(z8_hbm, c_stage)
    pltpu.sync_copy(c_stage, cnt_sh.at[pl.ds(base, EXP)])
    pltpu.sync_copy(ones_hbm, ones_v)
    plsc.subcore_barrier()
    rows_per_tile = (E // NW) // IB
    k = 8
    for it in range(rows_per_tile // k):
        roff = _m8((sc * NS + sid) * rows_per_tile + it * k)
        pltpu.sync_copy(col2d_hbm.at[pl.ds(roff, k)], idx_v)
        pltpu.sync_copy(h_hbm.at[pl.ds(_m8(roff * IB), k * IB)], h_v)
        for b in range(k):
            pltpu.sync_copy(h_v.at[pl.ds(b * IB, IB)],
                            acc_sh.at[idx_v.at[b]], add=True)
            pltpu.sync_copy(ones_v, cnt_sh.at[idx_v.at[b]], add=True)
    plsc.subcore_barrier()
    sl = pl.ds(base, EXP)
    pltpu.sync_copy(acc_sh.at[sl], ex_stage)
    pltpu.sync_copy(ex_stage, sum_hbm.at[sc, sl])
    pltpu.sync_copy(cnt_sh.at[sl], c_stage)
    pltpu.sync_copy(c_stage, cnt_hbm.at[sc, sl])


def _scatter_max_body(h_hbm, col_hbm, z128_hbm, maxp_hbm,
                      acc_v, idx0_v, idx1_v, h0_v, h1_v, dup_v, sem0, sem1):
    sc = lax.axis_index("c")
    sid = lax.axis_index("s")
    wid = sid * NC + sc
    g = wid % 4
    chk = wid // 4
    pltpu.sync_copy(z128_hbm, acc_v)
    iota16 = lax.broadcasted_iota(I32, (16,), 0)
    CH = 400
    per_chunk = E // 8
    ebase = chk * per_chunk
    nch = per_chunk // CH

    def fetch(it, idxr, hr, sem):
        off = _m8(ebase + it * CH)
        pltpu.async_copy(col_hbm.at[pl.ds(off, CH)], idxr, sem)
        pltpu.async_copy(h_hbm.at[pl.ds(off, CH), pl.ds(g * 8, 8)], hr, sem)

    def waitpair(it, idxr, hr, sem):
        off = _m8(ebase + it * CH)
        pltpu.make_async_copy(col_hbm.at[pl.ds(off, CH)], idxr, sem).wait()
        pltpu.make_async_copy(
            h_hbm.at[pl.ds(off, CH), pl.ds(g * 8, 8)], hr, sem).wait()

    def process(idxr, hr):
        def group_step(gi, carry2):
            e0 = gi * 16
            cols = idxr[pl.ds(e0, 16)]
            evec = e0 + iota16
            plsc.store_scatter(dup_v, [cols], iota16)
            rb = plsc.load_gather(dup_v, [cols])
            bv = jnp.right_shift(cols * 53688, 25)
            rowv = cols - bv * 625
            colb = jnp.left_shift(bv, 3)
            jvs = [jnp.full((16,), j, I32) for j in range(8)]
            cvs = [colb + j for j in range(8)]
            hvs = [plsc.load_gather(hr, [evec, jv]) for jv in jvs]
            curs = [plsc.load_gather(acc_v, [rowv, cv]) for cv in cvs]
            news = [jnp.maximum(c, v) for c, v in zip(curs, hvs)]
            for cv, nv in zip(cvs, news):
                plsc.store_scatter(acc_v, [rowv, cv], nv)
            ndup = plsc.all_reduce_population_count(rb != iota16)
            hasdup = jnp.squeeze(lax.slice(ndup, [0], [1]))

            @pl.when(hasdup > 0)
            def _fixup():
                for j in range(8):
                    cv = cvs[j]
                    hv = hvs[j]

                    def retry_cond(nbad):
                        return nbad > 0

                    def retry_body(nbad):
                        cur = plsc.load_gather(acc_v, [rowv, cv])
                        m = cur < hv
                        plsc.store_scatter(acc_v, [rowv, cv],
                                           jnp.maximum(cur, hv), mask=m)
                        cur2 = plsc.load_gather(acc_v, [rowv, cv])
                        return jnp.sum((cur2 < hv).astype(I32))

                    lax.while_loop(retry_cond, retry_body, jnp.int32(1))

            return carry2

        lax.fori_loop(0, CH // 16, group_step, 0)

    fetch(0, idx0_v, h0_v, sem0)

    def pair_step(itp, carry):
        it0 = 2 * itp
        it1 = it0 + 1
        waitpair(it0, idx0_v, h0_v, sem0)
        fetch(it1, idx1_v, h1_v, sem1)
        process(idx0_v, h0_v)
        waitpair(it1, idx1_v, h1_v, sem1)

        @pl.when(itp + 1 < nch // 2)
        def _pf():
            fetch(it0 + 2, idx0_v, h0_v, sem0)

        process(idx1_v, h1_v)
        return carry

    lax.fori_loop(0, nch // 2, pair_step, 0)
    pltpu.sync_copy(acc_v, maxp_hbm.at[chk, g])


def _final_body(sum_ref, cnt_ref, maxp_ref, base2_ref, w2m_ref, wsel_ref,
                w2b_ref, b2b_ref, out_ref):
    i = pl.program_id(0)
    s = sum_ref[0] + sum_ref[1]
    c8 = cnt_ref[0] + cnt_ref[1]
    cnt = jnp.maximum(c8[:, 0:1], 1.0)
    mean = s / cnt
    meanw = jnp.dot(mean, w2m_ref[...], preferred_element_type=F32)
    ms = []
    for g in range(4):
        m = maxp_ref[0, g]
        for c in range(1, 8):
            m = jnp.maximum(m, maxp_ref[c, g])
        ms.append(m)
    for k in range(8):
        kk = 8 * i + k
        mxw = jnp.dot(ms[0], wsel_ref[0, kk], preferred_element_type=F32)
        for g in range(1, 4):
            mxw = mxw + jnp.dot(ms[g], wsel_ref[g, kk],
                                preferred_element_type=F32)
        sl = pl.ds(k * 625, 625)
        h2 = jnp.maximum(
            base2_ref[sl, :] + meanw[k * 625:(k + 1) * 625, :] + mxw, 0.0)
        out_ref[sl, :] = jnp.maximum(
            jnp.dot(h2, w2b_ref[...], preferred_element_type=F32)
            + b2b_ref[...], 0.0)


def kernel(x, edge_index, edge_attr, u, batch,
           W1a, b1a, W1b, b1b, W2a, b2a, W2b, b2b):
    row = edge_index[0].astype(I32).reshape(50, 800, 8)
    col = edge_index[1].astype(I32).reshape(50, 800, 8)
    row = row.transpose(0, 2, 1).reshape(E)
    col = col.transpose(0, 2, 1).reshape(E)
    row2d = row.reshape(E // IB, IB)
    col2d = col.reshape(E // IB, IB)
    batch2d = batch.astype(I32).reshape(N, 1)
    W_rc = jnp.concatenate([W1a[:128], W1a[128:256]], axis=1)
    W1e = W1a[256:]
    W2x = W2a[:128]
    W2mean = W2a[128:160]
    W2max = W2a[160:192]
    W2u = W2a[192:224]
    b1a2 = b1a.reshape(1, 32)
    b1b2 = b1b.reshape(1, 32)
    b2a2 = b2a.reshape(1, 32)
    b2b2 = b2b.reshape(1, 128)
    ones8 = jnp.ones((IB, 8), F32)
    z32 = jnp.zeros((EXP, 32), F32)
    z8 = jnp.zeros((EXP, 8), F32)

    BN = 2000
    full = lambda shape: pl.BlockSpec(shape, lambda i, _s=shape: tuple(0 for _ in _s))
    xr, xc, base2 = pl.pallas_call(
        _node_proj_body,
        grid=(N // BN,),
        in_specs=[
            pl.BlockSpec((BN, 128), lambda i: (i, 0)),
            pl.BlockSpec((BN, 1), lambda i: (i, 0)),
            full((128, 64)), full((128, 32)), full((16, 32)),
            full((32, 32)), full((1, 32)),
        ],
        out_specs=[pl.BlockSpec((BN, 32), lambda i: (i, 0))] * 3,
        out_shape=[jax.ShapeDtypeStruct((N, 32), F32)] * 3,
    )(x, batch2d, W_rc, W2x, u, W2u, b2a2)

    mesh = plsc.VectorSubcoreMesh(core_axis_name="c", subcore_axis_name="s",
                                  num_cores=NC, num_subcores=NS)
    sc_params = pltpu.CompilerParams(use_tc_tiling_on_sc=False,
                                     needs_layout_passes=False)
    t_rc = pl.kernel(
        _gather_body,
        out_type=jax.ShapeDtypeStruct((E, 32), F32),
        mesh=mesh,
        compiler_params=sc_params,
        scratch_types=[
            pltpu.VMEM((8, IB), I32), pltpu.VMEM((8, IB), I32),
            pltpu.VMEM((1000, 32), F32), pltpu.VMEM((1000, 32), F32),
            pltpu.SemaphoreType.DMA, pltpu.SemaphoreType.DMA,
        ],
    )(xr, xc, row2d, col2d)

    BE = 6400
    ea8 = edge_attr.reshape(E // 8, 128)
    W1e_bd = jnp.zeros((128, 256), F32)
    for i in range(8):
        W1e_bd = W1e_bd.at[i * 16:(i + 1) * 16, i * 32:(i + 1) * 32].set(W1e)
    h = pl.pallas_call(
        _edge_mlp_body,
        grid=(E // BE,),
        in_specs=[
            pl.BlockSpec((BE, 32), lambda i: (i, 0)),
            pl.BlockSpec((BE // 8, 128), lambda i: (i, 0)),
            full((128, 256)), full((1, 32)), full((32, 32)), full((1, 32)),
        ],
        out_specs=pl.BlockSpec((BE, 32), lambda i: (i, 0)),
        out_shape=jax.ShapeDtypeStruct((E, 32), F32),
    )(t_rc, ea8, W1e_bd, b1a2, W1b, b1b2)

    sum_part, cnt_part = pl.kernel(
        _scatter_sum_body,
        out_type=[jax.ShapeDtypeStruct((NC, N, 32), F32),
                  jax.ShapeDtypeStruct((NC, N, 8), F32)],
        mesh=mesh,
        compiler_params=sc_params,
        scratch_types=[
            pltpu.VMEM((8, IB), I32),
            pltpu.VMEM((1000, 32), F32),
            pltpu.VMEM((IB, 8), F32),
            pltpu.VMEM((EXP, 8), F32),
            pltpu.VMEM((EXP, 32), F32),
            pltpu.VMEM_SHARED((N, 32), F32),
            pltpu.VMEM_SHARED((N, 8), F32),
        ],
    )(h, col2d, ones8, z32, z8)

    z128 = jnp.zeros((N // 16, 128), F32)
    max_part = pl.kernel(
        _scatter_max_body,
        out_type=jax.ShapeDtypeStruct((8, 4, N // 16, 128), F32),
        mesh=mesh,
        compiler_params=sc_params,
        scratch_types=[
            pltpu.VMEM((N // 16, 128), F32),
            pltpu.VMEM((400,), I32), pltpu.VMEM((400,), I32),
            pltpu.VMEM((400, 8), F32), pltpu.VMEM((400, 8), F32),
            pltpu.VMEM((N,), I32),
            pltpu.SemaphoreType.DMA, pltpu.SemaphoreType.DMA,
        ],
    )(h, col, z128)

    Wsel = jnp.einsum('kb,gjo->gkbjo', jnp.eye(16, dtype=F32),
                      W2max.reshape(4, 8, 32)).reshape(4, 16, 128, 32)
    BF = 5000
    out = pl.pallas_call(
        _final_body,
        grid=(N // BF,),
        in_specs=[
            pl.BlockSpec((NC, BF, 32), lambda i: (0, i, 0)),
            pl.BlockSpec((NC, BF, 8), lambda i: (0, i, 0)),
            pl.BlockSpec((8, 4, N // 16, 128), lambda i: (0, 0, 0, 0)),
            pl.BlockSpec((BF, 32), lambda i: (i, 0)),
            full((32, 32)), full((4, 16, 128, 32)), full((32, 128)),
            full((1, 128)),
        ],
        out_specs=pl.BlockSpec((BF, 128), lambda i: (i, 0)),
        out_shape=jax.ShapeDtypeStruct((N, 128), F32),
    )(sum_part, cnt_part, max_part, base2, W2mean, Wsel, W2b, b2b2)
    return out
